# baseline (speedup 1.0000x reference)
import functools

import jax
import jax.numpy as jnp
from jax import lax
from jax.experimental import pallas as pl
from jax.experimental.pallas import tpu as pltpu
from jax.experimental.pallas import tpu_sc as plsc

DIM = 64
NUM_CORES = 2
NUM_SUBCORES = 16
NUM_WORKERS = NUM_CORES * NUM_SUBCORES
CHUNK = 128


def _gather_kernel(n_total):
    b_per_w = n_total // NUM_WORKERS
    n_chunks = b_per_w // CHUNK
    mesh = plsc.VectorSubcoreMesh(
        core_axis_name="c",
        subcore_axis_name="s",
        num_cores=NUM_CORES,
        num_subcores=NUM_SUBCORES,
    )

    @functools.partial(
        pl.kernel,
        out_type=jax.ShapeDtypeStruct((n_total, DIM), jnp.float32),
        mesh=mesh,
        scratch_types=[
            pltpu.VMEM((n_chunks, CHUNK), jnp.int32),
            pltpu.VMEM((CHUNK,), jnp.int32),
            pltpu.VMEM((2, CHUNK, 2 * DIM), jnp.float32),
            pltpu.VMEM((CHUNK, DIM), jnp.float32),
            pltpu.SemaphoreType.DMA,
            pltpu.SemaphoreType.DMA,
        ],
        compiler_params=pltpu.CompilerParams(use_tc_tiling_on_sc=True),
    )
    def kern(idx_hbm, table_hbm, out_hbm, idx_v, pair_v, rows_v, sel_v, gsem, osem):
        wid = lax.axis_index("s") * NUM_CORES + lax.axis_index("c")
        base = wid * b_per_w
        pltpu.sync_copy(idx_hbm.at[wid], idx_v)

        def body(c, _):
            buf = lax.rem(c, 2)
            for v in range(CHUNK // 16):
                sl = pl.ds(v * 16, 16)
                pair_v[sl] = jax.lax.shift_right_logical(idx_v[c, sl], 1)
            pltpu.async_copy(table_hbm.at[pair_v], rows_v.at[buf], gsem).wait()
            for r0 in range(0, CHUNK, 16):
                hv = (idx_v[c, pl.ds(r0, 16)] & 1) * DIM
                for j in range(16):
                    off = hv[j]
                    for f0 in range(0, DIM, 16):
                        sel_v[r0 + j, pl.ds(f0, 16)] = rows_v[
                            buf, r0 + j, pl.ds(off + f0, 16)
                        ]
            pltpu.async_copy(
                sel_v,
                out_hbm.at[pl.ds(base + c * CHUNK, CHUNK)],
                osem,
            ).wait()
            return 0

        lax.fori_loop(0, n_chunks, body, 0, unroll=False)

    return kern


def kernel(source, W):
    n_total = source.shape[0] * source.shape[1]
    b_per_w = n_total // NUM_WORKERS
    table = W.reshape(500000, 2 * DIM)
    idx = source.reshape(NUM_WORKERS, b_per_w // CHUNK, CHUNK).astype(jnp.int32)
    out = _gather_kernel(n_total)(idx, table)
    return out.reshape(source.shape[0], source.shape[1], DIM)


# speedup vs baseline: 1.3673x; 1.3673x over previous
import functools

import jax
import jax.numpy as jnp
from jax import lax
from jax.experimental import pallas as pl
from jax.experimental.pallas import tpu as pltpu
from jax.experimental.pallas import tpu_sc as plsc

DIM = 64
NUM_CORES = 2
NUM_SUBCORES = 16
NUM_WORKERS = NUM_CORES * NUM_SUBCORES
CHUNK = 128


def _gather_kernel(n_total):
    b_per_w = n_total // NUM_WORKERS
    n_chunks = b_per_w // CHUNK
    mesh = plsc.VectorSubcoreMesh(
        core_axis_name="c",
        subcore_axis_name="s",
        num_cores=NUM_CORES,
        num_subcores=NUM_SUBCORES,
    )

    @functools.partial(
        pl.kernel,
        out_type=jax.ShapeDtypeStruct((n_total, DIM), jnp.float32),
        mesh=mesh,
        scratch_types=[
            pltpu.VMEM((n_chunks, CHUNK), jnp.int32),
            pltpu.VMEM((2, CHUNK), jnp.int32),
            pltpu.VMEM((2, CHUNK, 2 * DIM), jnp.float32),
            pltpu.VMEM((2, CHUNK, DIM), jnp.float32),
            pltpu.SemaphoreType.DMA,
            pltpu.SemaphoreType.DMA,
        ],
        compiler_params=pltpu.CompilerParams(use_tc_tiling_on_sc=True),
    )
    def kern(idx_hbm, table_hbm, out_hbm, idx_v, pair_v, rows_v, sel_v, gsem, osem):
        wid = lax.axis_index("s") * NUM_CORES + lax.axis_index("c")
        base = wid * b_per_w
        pltpu.sync_copy(idx_hbm.at[wid], idx_v)

        def gather_start(c, buf):
            for v in range(CHUNK // 16):
                sl = pl.ds(v * 16, 16)
                pair_v[buf, sl] = jax.lax.shift_right_logical(idx_v[c, sl], 1)
            pltpu.async_copy(table_hbm.at[pair_v.at[buf]], rows_v.at[buf], gsem)

        def gather_wait(buf):
            pltpu.make_async_copy(
                table_hbm.at[pair_v.at[buf]], rows_v.at[buf], gsem
            ).wait()

        def out_start(c, buf):
            pltpu.async_copy(
                sel_v.at[buf], out_hbm.at[pl.ds(base + c * CHUNK, CHUNK)], osem
            )

        def out_wait(c, buf):
            pltpu.make_async_copy(
                sel_v.at[buf], out_hbm.at[pl.ds(base + c * CHUNK, CHUNK)], osem
            ).wait()

        gather_start(0, 0)

        def body(c, _):
            buf = lax.rem(c, 2)
            nxt = 1 - buf
            gather_wait(buf)

            @pl.when(c + 1 < n_chunks)
            def _():
                gather_start(c + 1, nxt)

            @pl.when(c >= 2)
            def _():
                out_wait(c - 2, buf)

            for r0 in range(0, CHUNK, 16):
                hv = (idx_v[c, pl.ds(r0, 16)] & 1) * DIM
                for j in range(16):
                    off = hv[j]
                    for f0 in range(0, DIM, 16):
                        sel_v[buf, r0 + j, pl.ds(f0, 16)] = rows_v[
                            buf, r0 + j, pl.ds(off + f0, 16)
                        ]
            out_start(c, buf)
            return 0

        lax.fori_loop(0, n_chunks, body, 0, unroll=False)
        out_wait(n_chunks - 2, (n_chunks - 2) % 2)
        out_wait(n_chunks - 1, (n_chunks - 1) % 2)

    return kern


def kernel(source, W):
    n_total = source.shape[0] * source.shape[1]
    b_per_w = n_total // NUM_WORKERS
    table = W.reshape(500000, 2 * DIM)
    idx = source.reshape(NUM_WORKERS, b_per_w // CHUNK, CHUNK).astype(jnp.int32)
    out = _gather_kernel(n_total)(idx, table)
    return out.reshape(source.shape[0], source.shape[1], DIM)
